# baseline (speedup 1.0000x reference)
import functools

import jax
import jax.numpy as jnp
import numpy as np
from jax.experimental import pallas as pl
from jax.experimental.pallas import tpu as pltpu

D = 150
RADIUS = 3
ATOM_F = 39
BOND_F = 10
NFP = 5
BB = 2


def _lrelu(x):
    return jnp.where(x >= 0, x, 0.01 * x)


def _elu(x):
    return jnp.where(x > 0, x, jnp.exp(x) - 1.0)


def _pack_params(params):
    fps = [params["shared"], params["task1"], params["task2"],
           params["gate1"]["fp"], params["gate2"]["fp"]]

    def st(fn):
        return jnp.stack([fn(p) for p in fps])

    def str_(fn):
        return jnp.stack([fn(p, r) for p in fps for r in range(RADIUS)])

    pk = {}
    pk["wa"] = st(lambda p: p["atom_fc"]["W"].T)
    pk["ba"] = st(lambda p: p["atom_fc"]["b"])
    pk["wnba"] = st(lambda p: p["neighbor_fc"]["W"][:, :ATOM_F].T)
    pk["wnbb"] = st(lambda p: p["neighbor_fc"]["W"][:, ATOM_F:].T)
    pk["bnb"] = st(lambda p: p["neighbor_fc"]["b"])

    pk["al_wa"] = str_(lambda p, r: p["align"][r]["W"][0, :D, None])
    pk["al_wn"] = str_(lambda p, r: p["align"][r]["W"][0, D:, None])
    pk["wat"] = str_(lambda p, r: p["attend"][r]["W"].T)
    pk["bat"] = str_(lambda p, r: p["attend"][r]["b"])
    for i, g in enumerate(("r", "z", "n")):
        pk["gwi" + g] = str_(lambda p, r: p["gru"][r]["Wih"][i * D:(i + 1) * D].T)
        pk["gwh" + g] = str_(lambda p, r: p["gru"][r]["Whh"][i * D:(i + 1) * D].T)
        pk["gbi" + g] = str_(lambda p, r: p["gru"][r]["bih"][i * D:(i + 1) * D])
        pk["gbh" + g] = str_(lambda p, r: p["gru"][r]["bhh"][i * D:(i + 1) * D])

    pk["mwa"] = st(lambda p: p["mol_align"]["W"][0, :D, None])
    pk["mwn"] = st(lambda p: p["mol_align"]["W"][0, D:, None])
    pk["mwat"] = st(lambda p: p["mol_attend"]["W"].T)
    pk["mbat"] = st(lambda p: p["mol_attend"]["b"])
    for i, g in enumerate(("r", "z", "n")):
        pk["mgwi" + g] = st(lambda p: p["mol_gru"]["Wih"][i * D:(i + 1) * D].T)
        pk["mgwh" + g] = st(lambda p: p["mol_gru"]["Whh"][i * D:(i + 1) * D].T)
        pk["mgbi" + g] = st(lambda p: p["mol_gru"]["bih"][i * D:(i + 1) * D])
        pk["mgbh" + g] = st(lambda p: p["mol_gru"]["bhh"][i * D:(i + 1) * D])

    pk["gdw"] = jnp.stack([params["gate1"]["dnn"]["W"].T,
                           params["gate2"]["dnn"]["W"].T])
    pk["tw1"] = jnp.stack([params["tower1"]["fc1"]["W"].T,
                           params["tower2"]["fc1"]["W"].T])
    pk["tw2"] = jnp.stack([params["tower1"]["fc2"]["W"].T,
                           params["tower2"]["fc2"]["W"].T])
    pk["tb1"] = jnp.stack([params["tower1"]["fc1"]["b"],
                           params["tower2"]["fc1"]["b"]])

    sb = jnp.zeros((8, 128), dtype=jnp.float32)
    al_b = jnp.stack([p["align"][r]["b"][0] for p in fps for r in range(RADIUS)])
    sb = sb.at[0, :15].set(al_b)
    sb = sb.at[1, :5].set(jnp.stack([p["mol_align"]["b"][0] for p in fps]))
    sb = sb.at[2, :2].set(params["gate1"]["dnn"]["b"])
    sb = sb.at[2, 2:4].set(params["gate2"]["dnn"]["b"])
    sb = sb.at[3, 0].set(params["tower1"]["fc2"]["b"][0])
    sb = sb.at[3, 1].set(params["tower2"]["fc2"]["b"][0])
    pk["sbank"] = sb
    return pk


_WEIGHT_KEYS = ["wa", "ba", "wnba", "wnbb", "bnb", "al_wa", "al_wn", "wat",
                "bat",
                "gwir", "gwiz", "gwin", "gwhr", "gwhz", "gwhn",
                "gbir", "gbiz", "gbin", "gbhr", "gbhz", "gbhn",
                "mwa", "mwn", "mwat", "mbat",
                "mgwir", "mgwiz", "mgwin", "mgwhr", "mgwhz", "mgwhn",
                "mgbir", "mgbiz", "mgbin", "mgbhr", "mgbhz", "mgbhn",
                "gdw", "tw1", "tw2", "tb1", "sbank"]


def _dot(a, b):
    return jnp.dot(a, b, preferred_element_type=jnp.float32)


def _dotx(a, b):
    return jnp.dot(a, b, preferred_element_type=jnp.float32,
                   precision=jax.lax.Precision.HIGHEST)


def _sum_chunks(xs):
    return functools.reduce(lambda a, b: a + b, xs)


def _matic_kernel(L, NB, M,
                  atoms_ref, bonds_ref, adeg_ref, bdeg_ref, mask_ref,
                  wa, ba, wnba, wnbb, bnb, al_wa, al_wn, wat, bat,
                  gwir, gwiz, gwin, gwhr, gwhz, gwhn,
                  gbir, gbiz, gbin, gbhr, gbhz, gbhn,
                  mwa, mwn, mwat, mbat,
                  mgwir, mgwiz, mgwin, mgwhr, mgwhz, mgwhn,
                  mgbir, mgbiz, mgbin, mgbhr, mgbhz, mgbhn,
                  gdw, tw1, tw2, tb1, sbank,
                  out_ref, satt_ref, t1att_ref, t2att_ref, sel1_ref, sel2_ref,
                  sf1_ref, t1f1_ref, t2f1_ref, sf2_ref, t1f2_ref, t2f2_ref):
    f32 = jnp.float32
    atoms = atoms_ref[...]
    bonds = bonds_ref[...]
    adeg = adeg_ref[...]
    bdeg = bdeg_ref[...]
    mask = mask_ref[...]

    def gru(x, h, wir, wiz, win, whr, whz, whn, bir, biz, bin_, bhr, bhz, bhn):
        r = jax.nn.sigmoid(_dot(x, wir) + _dot(h, whr) + (bir + bhr))
        z = jax.nn.sigmoid(_dot(x, wiz) + _dot(h, whz) + (biz + bhz))
        n = jnp.tanh(_dot(x, win) + bin_ + r * (_dot(h, whn) + bhn))
        return (1.0 - z) * n + z * h

    iota_a = jax.lax.broadcasted_iota(jnp.int32, (L, L), 1)
    iota_b = jax.lax.broadcasted_iota(jnp.int32, (L, NB), 1)
    Ga = []
    Gb = []
    amask = []
    smask = []
    for mi in range(BB):
        ad = adeg[mi * L:(mi + 1) * L]
        bd = bdeg[mi * L:(mi + 1) * L]
        Ga.append(jnp.concatenate(
            [(ad[:, m:m + 1] == iota_a).astype(f32) for m in range(M)], axis=0))
        Gb.append(jnp.concatenate(
            [(bd[:, m:m + 1] == iota_b).astype(f32) for m in range(M)], axis=0))
        hit = jnp.concatenate([(ad[:, m:m + 1] == L - 1) for m in range(M)], axis=0)
        amask.append(jnp.where(hit, 0.0, 1.0))
        smask.append(jnp.where(hit, -9e8, 0.0))

    rawa = jnp.concatenate(
        [_dotx(Ga[mi], atoms[mi * L:(mi + 1) * L]) for mi in range(BB)], axis=0)
    rawb = jnp.concatenate(
        [_dotx(Gb[mi], bonds[mi * NB:(mi + 1) * NB]) for mi in range(BB)], axis=0)

    mol_smask = jnp.where(mask == 0.0, -9e8, 0.0)

    def softmax_m(sc):
        chunks = [sc[m * L:(m + 1) * L] for m in range(M)]
        mx = functools.reduce(jnp.maximum, chunks)
        es = [jnp.exp(c - mx) for c in chunks]
        tot = _sum_chunks(es)
        return [e / tot for e in es]

    def attention(k, h, act, nbf):
        al_b = sbank[0:1, k:k + 1]
        afs = _dot(h, al_wa[k]) + al_b
        if nbf is not None:
            nbs_all = _dot(nbf, al_wn[k])
            nft_all = _dot(nbf, wat[k]) + bat[k:k + 1, :]
        else:
            pa = _dot(act, al_wn[k])
            pat = _dot(act, wat[k]) + bat[k:k + 1, :]
        ctx = []
        for mi in range(BB):
            if nbf is not None:
                nbs = nbs_all[mi * M * L:(mi + 1) * M * L]
                nft = nft_all[mi * M * L:(mi + 1) * M * L]
            else:
                nbs = _dotx(Ga[mi], pa[mi * L:(mi + 1) * L])
                nft = _dotx(Ga[mi], pat[mi * L:(mi + 1) * L])
            afs_m = afs[mi * L:(mi + 1) * L]
            sc = _lrelu(jnp.concatenate([afs_m] * M, axis=0) + nbs) + smask[mi]
            ws = [w * amask[mi][m * L:(m + 1) * L]
                  for m, w in enumerate(softmax_m(sc))]
            ctx.append(_sum_chunks(
                [ws[m] * nft[m * L:(m + 1) * L] for m in range(M)]))
        return jnp.concatenate(ctx, axis=0)

    mol_feats = []
    for f in range(NFP):
        af = _lrelu(_dot(atoms, wa[f]) + ba[f:f + 1, :])
        nbf = _lrelu(_dot(rawa, wnba[f]) + _dot(rawb, wnbb[f]) +
                     bnb[f:f + 1, :])
        h = af
        act = None
        for d in range(RADIUS):
            k = f * RADIUS + d
            ctx = _elu(attention(k, h, act, nbf if d == 0 else None))
            h = gru(ctx, h, gwir[k], gwiz[k], gwin[k], gwhr[k], gwhz[k], gwhn[k],
                    gbir[k:k + 1, :], gbiz[k:k + 1, :], gbin[k:k + 1, :],
                    gbhr[k:k + 1, :], gbhz[k:k + 1, :], gbhn[k:k + 1, :])
            act = jnp.maximum(h, 0.0)

        masked_act = act * mask
        c1_in = []
        for mi in range(BB):
            mf = jnp.sum(masked_act[mi * L:(mi + 1) * L], axis=0, keepdims=True)
            c1_in.append(mf)
        mol_feature = jnp.concatenate(c1_in, axis=0)
        act_mol = jnp.maximum(mol_feature, 0.0)
        mb = sbank[1:2, f:f + 1]
        c1 = _dot(act_mol, mwa[f]) + mb
        s2 = _dot(act, mwn[f])
        aft = _dot(act, mwat[f]) + mbat[f:f + 1, :]
        maw_l = []
        mol_ctx_l = []
        for mi in range(BB):
            mas = _lrelu(c1[mi:mi + 1] + s2[mi * L:(mi + 1) * L]) \
                + mol_smask[mi * L:(mi + 1) * L]
            mmx = jnp.max(mas, axis=0, keepdims=True)
            me = jnp.exp(mas - mmx)
            maw = me / jnp.sum(me, axis=0, keepdims=True) * mask[mi * L:(mi + 1) * L]
            maw_l.append(maw)
            mol_ctx_l.append(jnp.sum(maw * aft[mi * L:(mi + 1) * L],
                                     axis=0, keepdims=True))
        maw = jnp.concatenate(maw_l, axis=0)
        mol_ctx = _elu(jnp.concatenate(mol_ctx_l, axis=0))
        mol_feature = gru(mol_ctx, mol_feature,
                          mgwir[f], mgwiz[f], mgwin[f],
                          mgwhr[f], mgwhz[f], mgwhn[f],
                          mgbir[f:f + 1, :], mgbiz[f:f + 1, :], mgbin[f:f + 1, :],
                          mgbhr[f:f + 1, :], mgbhz[f:f + 1, :], mgbhn[f:f + 1, :])
        mol_feats.append(mol_feature)

        if f == 0:
            satt_ref[...] = maw
            sf1_ref[...] = act
            sf2_ref[...] = h
        elif f == 1:
            t1att_ref[...] = maw
            t1f1_ref[...] = act
            t1f2_ref[...] = h
        elif f == 2:
            t2att_ref[...] = maw
            t2f1_ref[...] = act
            t2f2_ref[...] = h

    sels = []
    for g in range(2):
        logits = _dot(mol_feats[3 + g], gdw[g]) + sbank[2:3, 2 * g:2 * g + 2]
        mx = jnp.max(logits, axis=1, keepdims=True)
        e = jnp.exp(logits - mx)
        sels.append(e / jnp.sum(e, axis=1, keepdims=True))
    outs = []
    for g in range(2):
        gate_out = sels[g][:, 0:1] * mol_feats[1 + g] + sels[g][:, 1:2] * mol_feats[0]
        hdn = jnp.maximum(_dot(gate_out, tw1[g]) + tb1[g:g + 1, :], 0.0)
        outs.append(_dot(hdn, tw2[g]) + sbank[3:4, g:g + 1])

    out_ref[...] = jnp.concatenate(outs, axis=1).reshape(BB, 1, 2)
    sel1_ref[...] = sels[0].reshape(BB, 1, 2)
    sel2_ref[...] = sels[1].reshape(BB, 1, 2)


def kernel(atom_list, bond_list, atom_degree_list, bond_degree_list, atom_mask,
           params, interpret=False):
    B, L, _ = atom_list.shape
    NB = bond_list.shape[1]
    M = atom_degree_list.shape[-1]
    pk = _pack_params(params)

    atoms2 = atom_list.reshape(B * L, ATOM_F)
    bonds2 = bond_list.reshape(B * NB, BOND_F)
    adeg2 = atom_degree_list.astype(jnp.int32).reshape(B * L, M)
    bdeg2 = bond_degree_list.astype(jnp.int32).reshape(B * L, M)
    mask2 = atom_mask.reshape(B * L, 1)

    def full_spec(arr):
        r = arr.ndim
        return pl.BlockSpec(arr.shape, lambda i, _r=r: (0,) * _r)

    in_specs = [
        pl.BlockSpec((BB * L, ATOM_F), lambda i: (i, 0)),
        pl.BlockSpec((BB * NB, BOND_F), lambda i: (i, 0)),
        pl.BlockSpec((BB * L, M), lambda i: (i, 0)),
        pl.BlockSpec((BB * L, M), lambda i: (i, 0)),
        pl.BlockSpec((BB * L, 1), lambda i: (i, 0)),
    ] + [full_spec(pk[k]) for k in _WEIGHT_KEYS]

    def o2(shape, blk):
        return (jax.ShapeDtypeStruct(shape, jnp.float32),
                pl.BlockSpec(blk, (lambda i: (i, 0, 0)) if len(blk) == 3
                             else (lambda i: (i, 0))))

    out_shapes, out_specs = zip(*[
        o2((B, 1, 2), (BB, 1, 2)),
        o2((B * L, 1), (BB * L, 1)),
        o2((B * L, 1), (BB * L, 1)),
        o2((B * L, 1), (BB * L, 1)),
        o2((B, 1, 2), (BB, 1, 2)),
        o2((B, 1, 2), (BB, 1, 2)),
        o2((B * L, D), (BB * L, D)),
        o2((B * L, D), (BB * L, D)),
        o2((B * L, D), (BB * L, D)),
        o2((B * L, D), (BB * L, D)),
        o2((B * L, D), (BB * L, D)),
        o2((B * L, D), (BB * L, D)),
    ])

    fn = pl.pallas_call(
        functools.partial(_matic_kernel, L, NB, M),
        grid=(B // BB,),
        in_specs=list(in_specs),
        out_specs=list(out_specs),
        out_shape=list(out_shapes),
        interpret=interpret,
    )
    res = fn(atoms2, bonds2, adeg2, bdeg2, mask2,
             *[pk[k] for k in _WEIGHT_KEYS])
    (out, satt, t1att, t2att, sel1, sel2,
     sf1, t1f1, t2f1, sf2, t1f2, t2f2) = res
    out = out.reshape(B, 2)
    sel1 = sel1.reshape(B, 2)
    sel2 = sel2.reshape(B, 2)
    return (out,
            [satt.reshape(B, L, 1), t1att.reshape(B, L, 1),
             t2att.reshape(B, L, 1), sel1, sel2],
            [sf1.reshape(B, L, D), t1f1.reshape(B, L, D), t2f1.reshape(B, L, D)],
            [sf2.reshape(B, L, D), t1f2.reshape(B, L, D), t2f2.reshape(B, L, D)])


# speedup vs baseline: 1.8445x; 1.8445x over previous
import functools

import jax
import jax.numpy as jnp
import numpy as np
from jax.experimental import pallas as pl
from jax.experimental.pallas import tpu as pltpu

D = 150
RADIUS = 3
ATOM_F = 39
BOND_F = 10
NFP = 5
BB = 2


def _lrelu(x):
    return jnp.where(x >= 0, x, 0.01 * x)


def _elu(x):
    return jnp.where(x > 0, x, jnp.exp(x) - 1.0)


def _pack_params(params):
    fps = [params["shared"], params["task1"], params["task2"],
           params["gate1"]["fp"], params["gate2"]["fp"]]

    def st(fn):
        return jnp.stack([fn(p) for p in fps])

    def str_(fn):
        return jnp.stack([fn(p, r) for p in fps for r in range(RADIUS)])

    pk = {}
    pk["wa"] = st(lambda p: p["atom_fc"]["W"].T)
    pk["ba"] = st(lambda p: p["atom_fc"]["b"])
    pk["wnba"] = st(lambda p: p["neighbor_fc"]["W"][:, :ATOM_F].T)
    pk["wnbb"] = st(lambda p: p["neighbor_fc"]["W"][:, ATOM_F:].T)
    pk["bnb"] = st(lambda p: p["neighbor_fc"]["b"])

    pk["al_wa"] = str_(lambda p, r: p["align"][r]["W"][0, :D, None])
    pk["al_wn"] = str_(lambda p, r: p["align"][r]["W"][0, D:, None])
    pk["wat"] = str_(lambda p, r: p["attend"][r]["W"].T)
    pk["bat"] = str_(lambda p, r: p["attend"][r]["b"])
    for i, g in enumerate(("r", "z", "n")):
        pk["gwi" + g] = str_(lambda p, r: p["gru"][r]["Wih"][i * D:(i + 1) * D].T)
        pk["gwh" + g] = str_(lambda p, r: p["gru"][r]["Whh"][i * D:(i + 1) * D].T)
        pk["gbi" + g] = str_(lambda p, r: p["gru"][r]["bih"][i * D:(i + 1) * D])
        pk["gbh" + g] = str_(lambda p, r: p["gru"][r]["bhh"][i * D:(i + 1) * D])

    pk["mwa"] = st(lambda p: p["mol_align"]["W"][0, :D, None])
    pk["mwn"] = st(lambda p: p["mol_align"]["W"][0, D:, None])
    pk["mwat"] = st(lambda p: p["mol_attend"]["W"].T)
    pk["mbat"] = st(lambda p: p["mol_attend"]["b"])
    for i, g in enumerate(("r", "z", "n")):
        pk["mgwi" + g] = st(lambda p: p["mol_gru"]["Wih"][i * D:(i + 1) * D].T)
        pk["mgwh" + g] = st(lambda p: p["mol_gru"]["Whh"][i * D:(i + 1) * D].T)
        pk["mgbi" + g] = st(lambda p: p["mol_gru"]["bih"][i * D:(i + 1) * D])
        pk["mgbh" + g] = st(lambda p: p["mol_gru"]["bhh"][i * D:(i + 1) * D])

    pk["gdw"] = jnp.stack([params["gate1"]["dnn"]["W"].T,
                           params["gate2"]["dnn"]["W"].T])
    pk["tw1"] = jnp.stack([params["tower1"]["fc1"]["W"].T,
                           params["tower2"]["fc1"]["W"].T])
    pk["tw2"] = jnp.stack([params["tower1"]["fc2"]["W"].T,
                           params["tower2"]["fc2"]["W"].T])
    pk["tb1"] = jnp.stack([params["tower1"]["fc1"]["b"],
                           params["tower2"]["fc1"]["b"]])

    sb = jnp.zeros((8, 128), dtype=jnp.float32)
    al_b = jnp.stack([p["align"][r]["b"][0] for p in fps for r in range(RADIUS)])
    sb = sb.at[0, :15].set(al_b)
    sb = sb.at[1, :5].set(jnp.stack([p["mol_align"]["b"][0] for p in fps]))
    sb = sb.at[2, :2].set(params["gate1"]["dnn"]["b"])
    sb = sb.at[2, 2:4].set(params["gate2"]["dnn"]["b"])
    sb = sb.at[3, 0].set(params["tower1"]["fc2"]["b"][0])
    sb = sb.at[3, 1].set(params["tower2"]["fc2"]["b"][0])
    pk["sbank"] = sb
    return pk


_WEIGHT_KEYS = ["wa", "ba", "wnba", "wnbb", "bnb", "al_wa", "al_wn", "wat",
                "bat",
                "gwir", "gwiz", "gwin", "gwhr", "gwhz", "gwhn",
                "gbir", "gbiz", "gbin", "gbhr", "gbhz", "gbhn",
                "mwa", "mwn", "mwat", "mbat",
                "mgwir", "mgwiz", "mgwin", "mgwhr", "mgwhz", "mgwhn",
                "mgbir", "mgbiz", "mgbin", "mgbhr", "mgbhz", "mgbhn",
                "gdw", "tw1", "tw2", "tb1", "sbank"]


def _dot(a, b):
    return jnp.dot(a, b, preferred_element_type=jnp.float32)


def _dotx(a, b):
    return jnp.dot(a, b, preferred_element_type=jnp.float32,
                   precision=jax.lax.Precision.HIGHEST)


def _sum_chunks(xs):
    return functools.reduce(lambda a, b: a + b, xs)


def _matic_kernel(L, NB, M,
                  atoms_ref, bonds_ref, adeg_ref, bdeg_ref, mask_ref,
                  wa, ba, wnba, wnbb, bnb, al_wa, al_wn, wat, bat,
                  gwir, gwiz, gwin, gwhr, gwhz, gwhn,
                  gbir, gbiz, gbin, gbhr, gbhz, gbhn,
                  mwa, mwn, mwat, mbat,
                  mgwir, mgwiz, mgwin, mgwhr, mgwhz, mgwhn,
                  mgbir, mgbiz, mgbin, mgbhr, mgbhz, mgbhn,
                  gdw, tw1, tw2, tb1, sbank,
                  out_ref, satt_ref, t1att_ref, t2att_ref, sel1_ref, sel2_ref,
                  sf1_ref, t1f1_ref, t2f1_ref, sf2_ref, t1f2_ref, t2f2_ref):
    f32 = jnp.float32
    atoms = atoms_ref[...]
    bonds = bonds_ref[...]
    adeg = adeg_ref[...]
    bdeg = bdeg_ref[...]
    mask = mask_ref[...]

    def gru(x, h, wir, wiz, win, whr, whz, whn, bir, biz, bin_, bhr, bhz, bhn):
        r = jax.nn.sigmoid(_dot(x, wir) + _dot(h, whr) + (bir + bhr))
        z = jax.nn.sigmoid(_dot(x, wiz) + _dot(h, whz) + (biz + bhz))
        n = jnp.tanh(_dot(x, win) + bin_ + r * (_dot(h, whn) + bhn))
        return (1.0 - z) * n + z * h

    iota_a = jax.lax.broadcasted_iota(jnp.int32, (L, L), 1)
    iota_b = jax.lax.broadcasted_iota(jnp.int32, (L, NB), 1)
    Ga = []
    Gb = []
    amask = []
    smask = []
    for mi in range(BB):
        ad = adeg[mi * L:(mi + 1) * L]
        bd = bdeg[mi * L:(mi + 1) * L]
        Ga.append(jnp.concatenate(
            [(ad[:, m:m + 1] == iota_a).astype(f32) for m in range(M)], axis=0))
        Gb.append(jnp.concatenate(
            [(bd[:, m:m + 1] == iota_b).astype(f32) for m in range(M)], axis=0))
        hit = jnp.concatenate([(ad[:, m:m + 1] == L - 1) for m in range(M)], axis=0)
        amask.append(jnp.where(hit, 0.0, 1.0))
        smask.append(jnp.where(hit, -9e8, 0.0))

    rawa = jnp.concatenate(
        [_dot(Ga[mi], atoms[mi * L:(mi + 1) * L]) for mi in range(BB)], axis=0)
    rawb = jnp.concatenate(
        [_dot(Gb[mi], bonds[mi * NB:(mi + 1) * NB]) for mi in range(BB)], axis=0)

    mol_smask = jnp.where(mask == 0.0, -9e8, 0.0)

    def softmax_m(sc):
        chunks = [sc[m * L:(m + 1) * L] for m in range(M)]
        mx = functools.reduce(jnp.maximum, chunks)
        es = [jnp.exp(c - mx) for c in chunks]
        tot = _sum_chunks(es)
        return [e / tot for e in es]

    def attention(k, h, act, nbf):
        al_b = sbank[0:1, k:k + 1]
        afs = _dot(h, al_wa[k]) + al_b
        if nbf is not None:
            nbs_all = _dot(nbf, al_wn[k])
            nft_all = _dot(nbf, wat[k]) + bat[k:k + 1, :]
        else:
            pa = _dot(act, al_wn[k])
            pat = _dot(act, wat[k]) + bat[k:k + 1, :]
            pa_hi = pa.astype(jnp.bfloat16).astype(jnp.float32)
            pa_mid = pa - pa_hi
            pat_hi = pat.astype(jnp.bfloat16).astype(jnp.float32)
            pat_mid = pat - pat_hi
        ctx = []
        for mi in range(BB):
            if nbf is not None:
                nbs = nbs_all[mi * M * L:(mi + 1) * M * L]
                nft = nft_all[mi * M * L:(mi + 1) * M * L]
            else:
                sl = slice(mi * L, (mi + 1) * L)
                nbs = _dot(Ga[mi], pa_hi[sl]) + _dot(Ga[mi], pa_mid[sl])
                nft = _dot(Ga[mi], pat_hi[sl]) + _dot(Ga[mi], pat_mid[sl])
            afs_m = afs[mi * L:(mi + 1) * L]
            sc = _lrelu(jnp.concatenate([afs_m] * M, axis=0) + nbs) + smask[mi]
            ws = [w * amask[mi][m * L:(m + 1) * L]
                  for m, w in enumerate(softmax_m(sc))]
            ctx.append(_sum_chunks(
                [ws[m] * nft[m * L:(m + 1) * L] for m in range(M)]))
        return jnp.concatenate(ctx, axis=0)

    mol_feats = []
    for f in range(NFP):
        af = _lrelu(_dot(atoms, wa[f]) + ba[f:f + 1, :])
        nbf = _lrelu(_dot(rawa, wnba[f]) + _dot(rawb, wnbb[f]) +
                     bnb[f:f + 1, :])
        h = af
        act = None
        for d in range(RADIUS):
            k = f * RADIUS + d
            ctx = _elu(attention(k, h, act, nbf if d == 0 else None))
            h = gru(ctx, h, gwir[k], gwiz[k], gwin[k], gwhr[k], gwhz[k], gwhn[k],
                    gbir[k:k + 1, :], gbiz[k:k + 1, :], gbin[k:k + 1, :],
                    gbhr[k:k + 1, :], gbhz[k:k + 1, :], gbhn[k:k + 1, :])
            act = jnp.maximum(h, 0.0)

        masked_act = act * mask
        c1_in = []
        for mi in range(BB):
            mf = jnp.sum(masked_act[mi * L:(mi + 1) * L], axis=0, keepdims=True)
            c1_in.append(mf)
        mol_feature = jnp.concatenate(c1_in, axis=0)
        act_mol = jnp.maximum(mol_feature, 0.0)
        mb = sbank[1:2, f:f + 1]
        c1 = _dot(act_mol, mwa[f]) + mb
        s2 = _dot(act, mwn[f])
        aft = _dot(act, mwat[f]) + mbat[f:f + 1, :]
        maw_l = []
        mol_ctx_l = []
        for mi in range(BB):
            mas = _lrelu(c1[mi:mi + 1] + s2[mi * L:(mi + 1) * L]) \
                + mol_smask[mi * L:(mi + 1) * L]
            mmx = jnp.max(mas, axis=0, keepdims=True)
            me = jnp.exp(mas - mmx)
            maw = me / jnp.sum(me, axis=0, keepdims=True) * mask[mi * L:(mi + 1) * L]
            maw_l.append(maw)
            mol_ctx_l.append(jnp.sum(maw * aft[mi * L:(mi + 1) * L],
                                     axis=0, keepdims=True))
        maw = jnp.concatenate(maw_l, axis=0)
        mol_ctx = _elu(jnp.concatenate(mol_ctx_l, axis=0))
        mol_feature = gru(mol_ctx, mol_feature,
                          mgwir[f], mgwiz[f], mgwin[f],
                          mgwhr[f], mgwhz[f], mgwhn[f],
                          mgbir[f:f + 1, :], mgbiz[f:f + 1, :], mgbin[f:f + 1, :],
                          mgbhr[f:f + 1, :], mgbhz[f:f + 1, :], mgbhn[f:f + 1, :])
        mol_feats.append(mol_feature)

        if f == 0:
            satt_ref[...] = maw
            sf1_ref[...] = act
            sf2_ref[...] = h
        elif f == 1:
            t1att_ref[...] = maw
            t1f1_ref[...] = act
            t1f2_ref[...] = h
        elif f == 2:
            t2att_ref[...] = maw
            t2f1_ref[...] = act
            t2f2_ref[...] = h

    sels = []
    for g in range(2):
        logits = _dot(mol_feats[3 + g], gdw[g]) + sbank[2:3, 2 * g:2 * g + 2]
        mx = jnp.max(logits, axis=1, keepdims=True)
        e = jnp.exp(logits - mx)
        sels.append(e / jnp.sum(e, axis=1, keepdims=True))
    outs = []
    for g in range(2):
        gate_out = sels[g][:, 0:1] * mol_feats[1 + g] + sels[g][:, 1:2] * mol_feats[0]
        hdn = jnp.maximum(_dot(gate_out, tw1[g]) + tb1[g:g + 1, :], 0.0)
        outs.append(_dot(hdn, tw2[g]) + sbank[3:4, g:g + 1])

    out_ref[...] = jnp.concatenate(outs, axis=1).reshape(BB, 1, 2)
    sel1_ref[...] = sels[0].reshape(BB, 1, 2)
    sel2_ref[...] = sels[1].reshape(BB, 1, 2)


def kernel(atom_list, bond_list, atom_degree_list, bond_degree_list, atom_mask,
           params, interpret=False):
    B, L, _ = atom_list.shape
    NB = bond_list.shape[1]
    M = atom_degree_list.shape[-1]
    pk = _pack_params(params)

    atoms2 = atom_list.reshape(B * L, ATOM_F)
    bonds2 = bond_list.reshape(B * NB, BOND_F)
    adeg2 = atom_degree_list.astype(jnp.int32).reshape(B * L, M)
    bdeg2 = bond_degree_list.astype(jnp.int32).reshape(B * L, M)
    mask2 = atom_mask.reshape(B * L, 1)

    def full_spec(arr):
        r = arr.ndim
        return pl.BlockSpec(arr.shape, lambda i, _r=r: (0,) * _r)

    in_specs = [
        pl.BlockSpec((BB * L, ATOM_F), lambda i: (i, 0)),
        pl.BlockSpec((BB * NB, BOND_F), lambda i: (i, 0)),
        pl.BlockSpec((BB * L, M), lambda i: (i, 0)),
        pl.BlockSpec((BB * L, M), lambda i: (i, 0)),
        pl.BlockSpec((BB * L, 1), lambda i: (i, 0)),
    ] + [full_spec(pk[k]) for k in _WEIGHT_KEYS]

    def o2(shape, blk):
        return (jax.ShapeDtypeStruct(shape, jnp.float32),
                pl.BlockSpec(blk, (lambda i: (i, 0, 0)) if len(blk) == 3
                             else (lambda i: (i, 0))))

    out_shapes, out_specs = zip(*[
        o2((B, 1, 2), (BB, 1, 2)),
        o2((B * L, 1), (BB * L, 1)),
        o2((B * L, 1), (BB * L, 1)),
        o2((B * L, 1), (BB * L, 1)),
        o2((B, 1, 2), (BB, 1, 2)),
        o2((B, 1, 2), (BB, 1, 2)),
        o2((B * L, D), (BB * L, D)),
        o2((B * L, D), (BB * L, D)),
        o2((B * L, D), (BB * L, D)),
        o2((B * L, D), (BB * L, D)),
        o2((B * L, D), (BB * L, D)),
        o2((B * L, D), (BB * L, D)),
    ])

    fn = pl.pallas_call(
        functools.partial(_matic_kernel, L, NB, M),
        grid=(B // BB,),
        in_specs=list(in_specs),
        out_specs=list(out_specs),
        out_shape=list(out_shapes),
        interpret=interpret,
    )
    res = fn(atoms2, bonds2, adeg2, bdeg2, mask2,
             *[pk[k] for k in _WEIGHT_KEYS])
    (out, satt, t1att, t2att, sel1, sel2,
     sf1, t1f1, t2f1, sf2, t1f2, t2f2) = res
    out = out.reshape(B, 2)
    sel1 = sel1.reshape(B, 2)
    sel2 = sel2.reshape(B, 2)
    return (out,
            [satt.reshape(B, L, 1), t1att.reshape(B, L, 1),
             t2att.reshape(B, L, 1), sel1, sel2],
            [sf1.reshape(B, L, D), t1f1.reshape(B, L, D), t2f1.reshape(B, L, D)],
            [sf2.reshape(B, L, D), t1f2.reshape(B, L, D), t2f2.reshape(B, L, D)])


# speedup vs baseline: 1.8799x; 1.0191x over previous
import functools

import jax
import jax.numpy as jnp
import numpy as np
from jax.experimental import pallas as pl
from jax.experimental.pallas import tpu as pltpu

D = 150
RADIUS = 3
ATOM_F = 39
BOND_F = 10
NFP = 5
BB = 4


def _lrelu(x):
    return jnp.where(x >= 0, x, 0.01 * x)


def _elu(x):
    return jnp.where(x > 0, x, jnp.exp(x) - 1.0)


def _pack_params(params):
    fps = [params["shared"], params["task1"], params["task2"],
           params["gate1"]["fp"], params["gate2"]["fp"]]

    def st(fn):
        return jnp.stack([fn(p) for p in fps])

    def str_(fn):
        return jnp.stack([fn(p, r) for p in fps for r in range(RADIUS)])

    pk = {}
    pk["wa"] = st(lambda p: p["atom_fc"]["W"].T)
    pk["ba"] = st(lambda p: p["atom_fc"]["b"])
    pk["wnba"] = st(lambda p: p["neighbor_fc"]["W"][:, :ATOM_F].T)
    pk["wnbb"] = st(lambda p: p["neighbor_fc"]["W"][:, ATOM_F:].T)
    pk["bnb"] = st(lambda p: p["neighbor_fc"]["b"])

    pk["al_wa"] = str_(lambda p, r: p["align"][r]["W"][0, :D, None])
    pk["al_wn"] = str_(lambda p, r: p["align"][r]["W"][0, D:, None])
    pk["wat"] = str_(lambda p, r: p["attend"][r]["W"].T)
    pk["bat"] = str_(lambda p, r: p["attend"][r]["b"])
    for i, g in enumerate(("r", "z", "n")):
        pk["gwi" + g] = str_(lambda p, r: p["gru"][r]["Wih"][i * D:(i + 1) * D].T)
        pk["gwh" + g] = str_(lambda p, r: p["gru"][r]["Whh"][i * D:(i + 1) * D].T)
        pk["gbi" + g] = str_(lambda p, r: p["gru"][r]["bih"][i * D:(i + 1) * D])
        pk["gbh" + g] = str_(lambda p, r: p["gru"][r]["bhh"][i * D:(i + 1) * D])

    pk["mwa"] = st(lambda p: p["mol_align"]["W"][0, :D, None])
    pk["mwn"] = st(lambda p: p["mol_align"]["W"][0, D:, None])
    pk["mwat"] = st(lambda p: p["mol_attend"]["W"].T)
    pk["mbat"] = st(lambda p: p["mol_attend"]["b"])
    for i, g in enumerate(("r", "z", "n")):
        pk["mgwi" + g] = st(lambda p: p["mol_gru"]["Wih"][i * D:(i + 1) * D].T)
        pk["mgwh" + g] = st(lambda p: p["mol_gru"]["Whh"][i * D:(i + 1) * D].T)
        pk["mgbi" + g] = st(lambda p: p["mol_gru"]["bih"][i * D:(i + 1) * D])
        pk["mgbh" + g] = st(lambda p: p["mol_gru"]["bhh"][i * D:(i + 1) * D])

    pk["gdw"] = jnp.stack([params["gate1"]["dnn"]["W"].T,
                           params["gate2"]["dnn"]["W"].T])
    pk["tw1"] = jnp.stack([params["tower1"]["fc1"]["W"].T,
                           params["tower2"]["fc1"]["W"].T])
    pk["tw2"] = jnp.stack([params["tower1"]["fc2"]["W"].T,
                           params["tower2"]["fc2"]["W"].T])
    pk["tb1"] = jnp.stack([params["tower1"]["fc1"]["b"],
                           params["tower2"]["fc1"]["b"]])

    sb = jnp.zeros((8, 128), dtype=jnp.float32)
    al_b = jnp.stack([p["align"][r]["b"][0] for p in fps for r in range(RADIUS)])
    sb = sb.at[0, :15].set(al_b)
    sb = sb.at[1, :5].set(jnp.stack([p["mol_align"]["b"][0] for p in fps]))
    sb = sb.at[2, :2].set(params["gate1"]["dnn"]["b"])
    sb = sb.at[2, 2:4].set(params["gate2"]["dnn"]["b"])
    sb = sb.at[3, 0].set(params["tower1"]["fc2"]["b"][0])
    sb = sb.at[3, 1].set(params["tower2"]["fc2"]["b"][0])
    pk["sbank"] = sb
    return pk


_WEIGHT_KEYS = ["wa", "ba", "wnba", "wnbb", "bnb", "al_wa", "al_wn", "wat",
                "bat",
                "gwir", "gwiz", "gwin", "gwhr", "gwhz", "gwhn",
                "gbir", "gbiz", "gbin", "gbhr", "gbhz", "gbhn",
                "mwa", "mwn", "mwat", "mbat",
                "mgwir", "mgwiz", "mgwin", "mgwhr", "mgwhz", "mgwhn",
                "mgbir", "mgbiz", "mgbin", "mgbhr", "mgbhz", "mgbhn",
                "gdw", "tw1", "tw2", "tb1", "sbank"]


def _dot(a, b):
    return jnp.dot(a, b, preferred_element_type=jnp.float32)


def _dotx(a, b):
    return jnp.dot(a, b, preferred_element_type=jnp.float32,
                   precision=jax.lax.Precision.HIGHEST)


def _sum_chunks(xs):
    return functools.reduce(lambda a, b: a + b, xs)


def _matic_kernel(L, NB, M,
                  atoms_ref, bonds_ref, adeg_ref, bdeg_ref, mask_ref,
                  wa, ba, wnba, wnbb, bnb, al_wa, al_wn, wat, bat,
                  gwir, gwiz, gwin, gwhr, gwhz, gwhn,
                  gbir, gbiz, gbin, gbhr, gbhz, gbhn,
                  mwa, mwn, mwat, mbat,
                  mgwir, mgwiz, mgwin, mgwhr, mgwhz, mgwhn,
                  mgbir, mgbiz, mgbin, mgbhr, mgbhz, mgbhn,
                  gdw, tw1, tw2, tb1, sbank,
                  out_ref, satt_ref, t1att_ref, t2att_ref, sel1_ref, sel2_ref,
                  sf1_ref, t1f1_ref, t2f1_ref, sf2_ref, t1f2_ref, t2f2_ref):
    f32 = jnp.float32
    atoms = atoms_ref[...]
    bonds = bonds_ref[...]
    adeg = adeg_ref[...]
    bdeg = bdeg_ref[...]
    mask = mask_ref[...]

    def gru(x, h, wir, wiz, win, whr, whz, whn, bir, biz, bin_, bhr, bhz, bhn):
        r = jax.nn.sigmoid(_dot(x, wir) + _dot(h, whr) + (bir + bhr))
        z = jax.nn.sigmoid(_dot(x, wiz) + _dot(h, whz) + (biz + bhz))
        n = jnp.tanh(_dot(x, win) + bin_ + r * (_dot(h, whn) + bhn))
        return (1.0 - z) * n + z * h

    iota_a = jax.lax.broadcasted_iota(jnp.int32, (L, L), 1)
    iota_b = jax.lax.broadcasted_iota(jnp.int32, (L, NB), 1)
    Ga = []
    Gb = []
    amask = []
    smask = []
    for mi in range(BB):
        ad = adeg[mi * L:(mi + 1) * L]
        bd = bdeg[mi * L:(mi + 1) * L]
        Ga.append(jnp.concatenate(
            [(ad[:, m:m + 1] == iota_a).astype(f32) for m in range(M)], axis=0))
        Gb.append(jnp.concatenate(
            [(bd[:, m:m + 1] == iota_b).astype(f32) for m in range(M)], axis=0))
        hit = jnp.concatenate([(ad[:, m:m + 1] == L - 1) for m in range(M)], axis=0)
        amask.append(jnp.where(hit, 0.0, 1.0))
        smask.append(jnp.where(hit, -9e8, 0.0))

    rawa = jnp.concatenate(
        [_dot(Ga[mi], atoms[mi * L:(mi + 1) * L]) for mi in range(BB)], axis=0)
    rawb = jnp.concatenate(
        [_dot(Gb[mi], bonds[mi * NB:(mi + 1) * NB]) for mi in range(BB)], axis=0)

    mol_smask = jnp.where(mask == 0.0, -9e8, 0.0)

    def softmax_m(sc):
        chunks = [sc[m * L:(m + 1) * L] for m in range(M)]
        mx = functools.reduce(jnp.maximum, chunks)
        es = [jnp.exp(c - mx) for c in chunks]
        tot = _sum_chunks(es)
        return [e / tot for e in es]

    def attention(k, h, act, nbf):
        al_b = sbank[0:1, k:k + 1]
        afs = _dot(h, al_wa[k]) + al_b
        if nbf is not None:
            nbs_all = _dot(nbf, al_wn[k])
            nft_all = _dot(nbf, wat[k]) + bat[k:k + 1, :]
        else:
            pa = _dot(act, al_wn[k])
            pat = _dot(act, wat[k]) + bat[k:k + 1, :]
            pa_hi = pa.astype(jnp.bfloat16).astype(jnp.float32)
            pa_mid = pa - pa_hi
            pat_hi = pat.astype(jnp.bfloat16).astype(jnp.float32)
            pat_mid = pat - pat_hi
        ctx = []
        for mi in range(BB):
            if nbf is not None:
                nbs = nbs_all[mi * M * L:(mi + 1) * M * L]
                nft = nft_all[mi * M * L:(mi + 1) * M * L]
            else:
                sl = slice(mi * L, (mi + 1) * L)
                nbs = _dot(Ga[mi], pa_hi[sl]) + _dot(Ga[mi], pa_mid[sl])
                nft = _dot(Ga[mi], pat_hi[sl]) + _dot(Ga[mi], pat_mid[sl])
            afs_m = afs[mi * L:(mi + 1) * L]
            sc = _lrelu(jnp.concatenate([afs_m] * M, axis=0) + nbs) + smask[mi]
            ws = [w * amask[mi][m * L:(m + 1) * L]
                  for m, w in enumerate(softmax_m(sc))]
            ctx.append(_sum_chunks(
                [ws[m] * nft[m * L:(m + 1) * L] for m in range(M)]))
        return jnp.concatenate(ctx, axis=0)

    mol_feats = []
    for f in range(NFP):
        af = _lrelu(_dot(atoms, wa[f]) + ba[f:f + 1, :])
        nbf = _lrelu(_dot(rawa, wnba[f]) + _dot(rawb, wnbb[f]) +
                     bnb[f:f + 1, :])
        h = af
        act = None
        for d in range(RADIUS):
            k = f * RADIUS + d
            ctx = _elu(attention(k, h, act, nbf if d == 0 else None))
            h = gru(ctx, h, gwir[k], gwiz[k], gwin[k], gwhr[k], gwhz[k], gwhn[k],
                    gbir[k:k + 1, :], gbiz[k:k + 1, :], gbin[k:k + 1, :],
                    gbhr[k:k + 1, :], gbhz[k:k + 1, :], gbhn[k:k + 1, :])
            act = jnp.maximum(h, 0.0)

        masked_act = act * mask
        c1_in = []
        for mi in range(BB):
            mf = jnp.sum(masked_act[mi * L:(mi + 1) * L], axis=0, keepdims=True)
            c1_in.append(mf)
        mol_feature = jnp.concatenate(c1_in, axis=0)
        act_mol = jnp.maximum(mol_feature, 0.0)
        mb = sbank[1:2, f:f + 1]
        c1 = _dot(act_mol, mwa[f]) + mb
        s2 = _dot(act, mwn[f])
        aft = _dot(act, mwat[f]) + mbat[f:f + 1, :]
        maw_l = []
        mol_ctx_l = []
        for mi in range(BB):
            mas = _lrelu(c1[mi:mi + 1] + s2[mi * L:(mi + 1) * L]) \
                + mol_smask[mi * L:(mi + 1) * L]
            mmx = jnp.max(mas, axis=0, keepdims=True)
            me = jnp.exp(mas - mmx)
            maw = me / jnp.sum(me, axis=0, keepdims=True) * mask[mi * L:(mi + 1) * L]
            maw_l.append(maw)
            mol_ctx_l.append(jnp.sum(maw * aft[mi * L:(mi + 1) * L],
                                     axis=0, keepdims=True))
        maw = jnp.concatenate(maw_l, axis=0)
        mol_ctx = _elu(jnp.concatenate(mol_ctx_l, axis=0))
        mol_feature = gru(mol_ctx, mol_feature,
                          mgwir[f], mgwiz[f], mgwin[f],
                          mgwhr[f], mgwhz[f], mgwhn[f],
                          mgbir[f:f + 1, :], mgbiz[f:f + 1, :], mgbin[f:f + 1, :],
                          mgbhr[f:f + 1, :], mgbhz[f:f + 1, :], mgbhn[f:f + 1, :])
        mol_feats.append(mol_feature)

        if f == 0:
            satt_ref[...] = maw
            sf1_ref[...] = act
            sf2_ref[...] = h
        elif f == 1:
            t1att_ref[...] = maw
            t1f1_ref[...] = act
            t1f2_ref[...] = h
        elif f == 2:
            t2att_ref[...] = maw
            t2f1_ref[...] = act
            t2f2_ref[...] = h

    sels = []
    for g in range(2):
        logits = _dot(mol_feats[3 + g], gdw[g]) + sbank[2:3, 2 * g:2 * g + 2]
        mx = jnp.max(logits, axis=1, keepdims=True)
        e = jnp.exp(logits - mx)
        sels.append(e / jnp.sum(e, axis=1, keepdims=True))
    outs = []
    for g in range(2):
        gate_out = sels[g][:, 0:1] * mol_feats[1 + g] + sels[g][:, 1:2] * mol_feats[0]
        hdn = jnp.maximum(_dot(gate_out, tw1[g]) + tb1[g:g + 1, :], 0.0)
        outs.append(_dot(hdn, tw2[g]) + sbank[3:4, g:g + 1])

    out_ref[...] = jnp.concatenate(outs, axis=1).reshape(BB, 1, 2)
    sel1_ref[...] = sels[0].reshape(BB, 1, 2)
    sel2_ref[...] = sels[1].reshape(BB, 1, 2)


def kernel(atom_list, bond_list, atom_degree_list, bond_degree_list, atom_mask,
           params, interpret=False):
    B, L, _ = atom_list.shape
    NB = bond_list.shape[1]
    M = atom_degree_list.shape[-1]
    pk = _pack_params(params)

    atoms2 = atom_list.reshape(B * L, ATOM_F)
    bonds2 = bond_list.reshape(B * NB, BOND_F)
    adeg2 = atom_degree_list.astype(jnp.int32).reshape(B * L, M)
    bdeg2 = bond_degree_list.astype(jnp.int32).reshape(B * L, M)
    mask2 = atom_mask.reshape(B * L, 1)

    def full_spec(arr):
        r = arr.ndim
        return pl.BlockSpec(arr.shape, lambda i, _r=r: (0,) * _r)

    in_specs = [
        pl.BlockSpec((BB * L, ATOM_F), lambda i: (i, 0)),
        pl.BlockSpec((BB * NB, BOND_F), lambda i: (i, 0)),
        pl.BlockSpec((BB * L, M), lambda i: (i, 0)),
        pl.BlockSpec((BB * L, M), lambda i: (i, 0)),
        pl.BlockSpec((BB * L, 1), lambda i: (i, 0)),
    ] + [full_spec(pk[k]) for k in _WEIGHT_KEYS]

    def o2(shape, blk):
        return (jax.ShapeDtypeStruct(shape, jnp.float32),
                pl.BlockSpec(blk, (lambda i: (i, 0, 0)) if len(blk) == 3
                             else (lambda i: (i, 0))))

    out_shapes, out_specs = zip(*[
        o2((B, 1, 2), (BB, 1, 2)),
        o2((B * L, 1), (BB * L, 1)),
        o2((B * L, 1), (BB * L, 1)),
        o2((B * L, 1), (BB * L, 1)),
        o2((B, 1, 2), (BB, 1, 2)),
        o2((B, 1, 2), (BB, 1, 2)),
        o2((B * L, D), (BB * L, D)),
        o2((B * L, D), (BB * L, D)),
        o2((B * L, D), (BB * L, D)),
        o2((B * L, D), (BB * L, D)),
        o2((B * L, D), (BB * L, D)),
        o2((B * L, D), (BB * L, D)),
    ])

    fn = pl.pallas_call(
        functools.partial(_matic_kernel, L, NB, M),
        grid=(B // BB,),
        in_specs=list(in_specs),
        out_specs=list(out_specs),
        out_shape=list(out_shapes),
        interpret=interpret,
    )
    res = fn(atoms2, bonds2, adeg2, bdeg2, mask2,
             *[pk[k] for k in _WEIGHT_KEYS])
    (out, satt, t1att, t2att, sel1, sel2,
     sf1, t1f1, t2f1, sf2, t1f2, t2f2) = res
    out = out.reshape(B, 2)
    sel1 = sel1.reshape(B, 2)
    sel2 = sel2.reshape(B, 2)
    return (out,
            [satt.reshape(B, L, 1), t1att.reshape(B, L, 1),
             t2att.reshape(B, L, 1), sel1, sel2],
            [sf1.reshape(B, L, D), t1f1.reshape(B, L, D), t2f1.reshape(B, L, D)],
            [sf2.reshape(B, L, D), t1f2.reshape(B, L, D), t2f2.reshape(B, L, D)])
